# baseline (speedup 1.0000x reference)
import dataclasses

import jax
import jax.numpy as jnp
from jax.experimental import pallas as pl
from jax.experimental.pallas import tpu as pltpu
from jax.experimental.pallas import tpu_sc as plsc

_N = 68
_E = 544
_H = 128
_K = 16
_EPS = 1e-15
_P = 128
_CPC = 17


def _sc_build_a_body(src_hbm, dst_hbm, out_hbm, src_v, dst_v, oh_v, shared):
    c = jax.lax.axis_index("c")
    s = jax.lax.axis_index("s")
    zero16 = jnp.zeros((16,), jnp.float32)
    for r in range(16):
        for g in range(_P // 16):
            oh_v[r, pl.ds(16 * g, 16)] = zero16

    @pl.when(s < 4)
    def _():
        pltpu.sync_copy(oh_v, shared.at[pl.ds(s * 16, 16)])

    @pl.when(s == 4)
    def _():
        pltpu.sync_copy(oh_v.at[pl.ds(0, 4)], shared.at[pl.ds(64, 4)])

    plsc.subcore_barrier()

    def do_chunk(chunk):
        pltpu.sync_copy(src_hbm.at[chunk], src_v.at[0])
        pltpu.sync_copy(dst_hbm.at[chunk], dst_v.at[0])

        @pl.loop(0, 16)
        def _(k):
            kk = jnp.zeros((16,), jnp.int32) + k
            sv = plsc.load_gather(src_v, [jnp.zeros((16,), jnp.int32), kk])
            for g in range(_P // 16):
                seg = jax.lax.iota(jnp.int32, 16) + (16 * g)
                oh_v[k, pl.ds(16 * g, 16)] = (seg == sv).astype(jnp.float32)

        pltpu.sync_copy(oh_v, shared.at[dst_v.at[0]], add=True)

    do_chunk(c * _CPC + s)

    @pl.when(s == 0)
    def _():
        do_chunk(c * _CPC + 16)

    plsc.subcore_barrier()

    @pl.when(s < 4)
    def _():
        pltpu.sync_copy(shared.at[pl.ds(s * 16, 16)],
                        out_hbm.at[c].at[pl.ds(s * 16, 16)])

    @pl.when(s == 4)
    def _():
        pltpu.sync_copy(shared.at[pl.ds(64, 4)], out_hbm.at[c].at[pl.ds(64, 4)])


def _sc_build_a(src2d, dst2d):
    mesh = plsc.VectorSubcoreMesh(core_axis_name="c", subcore_axis_name="s")
    cp = pltpu.CompilerParams()
    if "needs_layout_passes" in pltpu.CompilerParams.__dataclass_fields__:
        cp = dataclasses.replace(cp, needs_layout_passes=False)
    f = pl.kernel(
        _sc_build_a_body,
        out_type=jax.ShapeDtypeStruct((2, _N, _P), jnp.float32),
        mesh=mesh,
        scratch_types=[
            pltpu.VMEM((1, 16), jnp.int32),
            pltpu.VMEM((1, 16), jnp.int32),
            pltpu.VMEM((16, _P), jnp.float32),
            pltpu.VMEM_SHARED((_N, _P), jnp.float32),
        ],
        compiler_params=cp,
    )
    return f(src2d, dst2d)


def _fused_body(a_ref, x_ref, adj_ref, s_ref,
                w1r_ref, b1_ref, w1s_ref,
                w2r_ref, b2_ref, w2s_ref,
                w3r_ref, b3_ref, w3s_ref,
                out_ref, oadj_ref, loss_ref):
    f32 = jnp.float32
    ap = a_ref[...]
    a_mat = (ap[0] + ap[1])[:, 0:_N]

    def layer(h, wr, b, wroot):
        rel = jax.lax.dot_general(h, wr, (((1,), (1,)), ((), ())),
                                  preferred_element_type=f32)
        agg = jnp.dot(a_mat, rel, preferred_element_type=f32)
        root = jax.lax.dot_general(h, wroot, (((1,), (1,)), ((), ())),
                                   preferred_element_type=f32)
        return jnp.maximum(agg + root + b, 0.0)

    h = layer(x_ref[...], w1r_ref[...], b1_ref[...], w1s_ref[...])
    h = layer(h, w2r_ref[...], b2_ref[...], w2s_ref[...])
    h = layer(h, w3r_ref[...], b3_ref[...], w3s_ref[...])

    s = s_ref[...]
    m = jnp.max(s, axis=1, keepdims=True)
    e = jnp.exp(s - m)
    ssm = e / jnp.sum(e, axis=1, keepdims=True)

    out_ref[...] = jax.lax.dot_general(ssm, h, (((0,), (0,)), ((), ())),
                                       preferred_element_type=f32)
    adj = adj_ref[...]
    sta = jax.lax.dot_general(ssm, adj, (((0,), (0,)), ((), ())),
                              preferred_element_type=f32)
    oadj_ref[...] = jnp.dot(sta, ssm, preferred_element_type=f32)

    sst = jax.lax.dot_general(ssm, ssm, (((1,), (1,)), ((), ())),
                              preferred_element_type=f32)
    link = adj - sst
    ll = jnp.sqrt(jnp.sum(link * link, keepdims=True)) / (_N * _N)
    ent = -jnp.sum(ssm * jnp.log(ssm + _EPS), keepdims=True) / _N
    loss_ref[...] = jnp.concatenate([ll, ent], axis=1)


def kernel(x, edge_index, pos, adj, s,
           W1_rel, b1, W1_root, W2_rel, b2, W2_root, W3_rel, b3, W3_root):
    src2d = edge_index[0].reshape(_E // 16, 16)
    dst2d = edge_index[1].reshape(_E // 16, 16)
    a_pad = _sc_build_a(src2d, dst2d)

    out, out_adj, losses = pl.pallas_call(
        _fused_body,
        out_shape=[
            jax.ShapeDtypeStruct((_K, _H), jnp.float32),
            jax.ShapeDtypeStruct((_K, _K), jnp.float32),
            jax.ShapeDtypeStruct((1, 2), jnp.float32),
        ],
    )(a_pad, x, adj.reshape(_N, _N), s.reshape(_N, _K),
      W1_rel, b1.reshape(1, _H), W1_root,
      W2_rel, b2.reshape(1, _H), W2_root,
      W3_rel, b3.reshape(1, _H), W3_root)
    return (out.reshape(1, _K, _H), out_adj.reshape(1, _K, _K),
            losses[0, 0], losses[0, 1], pos)


# speedup vs baseline: 1.0042x; 1.0042x over previous
import dataclasses

import jax
import jax.numpy as jnp
from jax.experimental import pallas as pl
from jax.experimental.pallas import tpu as pltpu
from jax.experimental.pallas import tpu_sc as plsc

_N = 68
_E = 544
_H = 128
_K = 16
_EPS = 1e-15
_P = 128
_CPC = 17


def _sc_build_a_body(edge_hbm, out_hbm, src_v, dst_v, oh_v, shared):
    c = jax.lax.axis_index("c")
    s = jax.lax.axis_index("s")
    zero16 = jnp.zeros((16,), jnp.float32)
    for r in range(16):
        for g in range(_P // 16):
            oh_v[r, pl.ds(16 * g, 16)] = zero16

    @pl.when(s < 4)
    def _():
        pltpu.sync_copy(oh_v, shared.at[pl.ds(s * 16, 16)])

    @pl.when(s == 4)
    def _():
        pltpu.sync_copy(oh_v.at[pl.ds(0, 4)], shared.at[pl.ds(64, 4)])

    plsc.subcore_barrier()

    def do_chunk(chunk):
        pltpu.sync_copy(edge_hbm.at[0].at[pl.ds(chunk * 16, 16)], src_v.at[0])
        pltpu.sync_copy(edge_hbm.at[1].at[pl.ds(chunk * 16, 16)], dst_v.at[0])

        @pl.loop(0, 16)
        def _(k):
            kk = jnp.zeros((16,), jnp.int32) + k
            sv = plsc.load_gather(src_v, [jnp.zeros((16,), jnp.int32), kk])
            for g in range(_P // 16):
                seg = jax.lax.iota(jnp.int32, 16) + (16 * g)
                oh_v[k, pl.ds(16 * g, 16)] = (seg == sv).astype(jnp.float32)

        pltpu.sync_copy(oh_v, shared.at[dst_v.at[0]], add=True)

    do_chunk(c * _CPC + s)

    @pl.when(s == 0)
    def _():
        do_chunk(c * _CPC + 16)

    plsc.subcore_barrier()

    @pl.when(s < 4)
    def _():
        pltpu.sync_copy(shared.at[pl.ds(s * 16, 16)],
                        out_hbm.at[c].at[pl.ds(s * 16, 16)])

    @pl.when(s == 4)
    def _():
        pltpu.sync_copy(shared.at[pl.ds(64, 4)], out_hbm.at[c].at[pl.ds(64, 4)])


def _sc_build_a(edge_index):
    mesh = plsc.VectorSubcoreMesh(core_axis_name="c", subcore_axis_name="s")
    cp = pltpu.CompilerParams()
    if "needs_layout_passes" in pltpu.CompilerParams.__dataclass_fields__:
        cp = dataclasses.replace(cp, needs_layout_passes=False)
    f = pl.kernel(
        _sc_build_a_body,
        out_type=jax.ShapeDtypeStruct((2, _N, _P), jnp.float32),
        mesh=mesh,
        scratch_types=[
            pltpu.VMEM((1, 16), jnp.int32),
            pltpu.VMEM((1, 16), jnp.int32),
            pltpu.VMEM((16, _P), jnp.float32),
            pltpu.VMEM_SHARED((_N, _P), jnp.float32),
        ],
        compiler_params=cp,
    )
    return f(edge_index)


def _fused_body(a_ref, x_ref, adj_ref, s_ref,
                w1r_ref, b1_ref, w1s_ref,
                w2r_ref, b2_ref, w2s_ref,
                w3r_ref, b3_ref, w3s_ref,
                out_ref, oadj_ref, loss_ref):
    f32 = jnp.float32
    ap = a_ref[...]
    a_mat = (ap[0] + ap[1])[:, 0:_N]

    def layer(h, wr, b, wroot):
        rel = jax.lax.dot_general(h, wr, (((1,), (1,)), ((), ())),
                                  preferred_element_type=f32)
        agg = jnp.dot(a_mat, rel, preferred_element_type=f32)
        root = jax.lax.dot_general(h, wroot, (((1,), (1,)), ((), ())),
                                   preferred_element_type=f32)
        return jnp.maximum(agg + root + b, 0.0)

    h = layer(x_ref[...], w1r_ref[...], b1_ref[...], w1s_ref[...])
    h = layer(h, w2r_ref[...], b2_ref[...], w2s_ref[...])
    h = layer(h, w3r_ref[...], b3_ref[...], w3s_ref[...])

    s = s_ref[...][0]
    m = jnp.max(s, axis=1, keepdims=True)
    e = jnp.exp(s - m)
    ssm = e / jnp.sum(e, axis=1, keepdims=True)

    out = jax.lax.dot_general(ssm, h, (((0,), (0,)), ((), ())),
                              preferred_element_type=f32)
    out_ref[...] = out.reshape(1, _K, _H)
    adj = adj_ref[...][0]
    sta = jax.lax.dot_general(ssm, adj, (((0,), (0,)), ((), ())),
                              preferred_element_type=f32)
    oadj = jnp.dot(sta, ssm, preferred_element_type=f32)
    oadj_ref[...] = oadj.reshape(1, _K, _K)

    sst = jax.lax.dot_general(ssm, ssm, (((1,), (1,)), ((), ())),
                              preferred_element_type=f32)
    link = adj - sst
    ll = jnp.sqrt(jnp.sum(link * link, keepdims=True)) / (_N * _N)
    ent = -jnp.sum(ssm * jnp.log(ssm + _EPS), keepdims=True) / _N
    loss_ref[...] = jnp.concatenate([ll, ent], axis=1)


def kernel(x, edge_index, pos, adj, s,
           W1_rel, b1, W1_root, W2_rel, b2, W2_root, W3_rel, b3, W3_root):
    a_pad = _sc_build_a(edge_index)

    out, out_adj, losses = pl.pallas_call(
        _fused_body,
        out_shape=[
            jax.ShapeDtypeStruct((1, _K, _H), jnp.float32),
            jax.ShapeDtypeStruct((1, _K, _K), jnp.float32),
            jax.ShapeDtypeStruct((1, 2), jnp.float32),
        ],
    )(a_pad, x, adj, s,
      W1_rel, b1, W1_root, W2_rel, b2, W2_root, W3_rel, b3, W3_root)
    return (out, out_adj, losses[0, 0], losses[0, 1], pos)


# speedup vs baseline: 1.0239x; 1.0196x over previous
import dataclasses

import jax
import jax.numpy as jnp
from jax.experimental import pallas as pl
from jax.experimental.pallas import tpu as pltpu
from jax.experimental.pallas import tpu_sc as plsc

_N = 68
_E = 544
_H = 128
_K = 16
_EPS = 1e-15
_P = 128
_CPC = 17


def _sc_build_a_body(edge_hbm, out_hbm, src_v, dst_v, oh_v, shared):
    c = jax.lax.axis_index("c")
    s = jax.lax.axis_index("s")
    zero16 = jnp.zeros((16,), jnp.float32)
    for r in range(16):
        for g in range(_P // 16):
            oh_v[r, pl.ds(16 * g, 16)] = zero16

    @pl.when(s < 4)
    def _():
        pltpu.sync_copy(oh_v, shared.at[pl.ds(s * 16, 16)])

    @pl.when(s == 4)
    def _():
        pltpu.sync_copy(oh_v.at[pl.ds(0, 4)], shared.at[pl.ds(64, 4)])

    plsc.subcore_barrier()

    def do_chunk(chunk):
        pltpu.sync_copy(edge_hbm.at[0].at[pl.ds(chunk * 16, 16)], src_v.at[0])
        pltpu.sync_copy(edge_hbm.at[1].at[pl.ds(chunk * 16, 16)], dst_v.at[0])

        @pl.loop(0, 16)
        def _(k):
            kk = jnp.zeros((16,), jnp.int32) + k
            sv = plsc.load_gather(src_v, [jnp.zeros((16,), jnp.int32), kk])
            for g in range(_P // 16):
                seg = jax.lax.iota(jnp.int32, 16) + (16 * g)
                oh_v[k, pl.ds(16 * g, 16)] = (seg == sv).astype(jnp.float32)

        pltpu.sync_copy(oh_v, shared.at[dst_v.at[0]], add=True)

    do_chunk(c * _CPC + s)

    @pl.when(s == 0)
    def _():
        do_chunk(c * _CPC + 16)

    plsc.subcore_barrier()

    @pl.when(s < 4)
    def _():
        pltpu.sync_copy(shared.at[pl.ds(s * 16, 16)],
                        out_hbm.at[c].at[pl.ds(s * 16, 16)])

    @pl.when(s == 4)
    def _():
        pltpu.sync_copy(shared.at[pl.ds(64, 4)], out_hbm.at[c].at[pl.ds(64, 4)])


def _sc_build_a(edge_index):
    mesh = plsc.VectorSubcoreMesh(core_axis_name="c", subcore_axis_name="s")
    cp = pltpu.CompilerParams()
    if "needs_layout_passes" in pltpu.CompilerParams.__dataclass_fields__:
        cp = dataclasses.replace(cp, needs_layout_passes=False)
    f = pl.kernel(
        _sc_build_a_body,
        out_type=jax.ShapeDtypeStruct((2, _N, _P), jnp.float32),
        mesh=mesh,
        scratch_types=[
            pltpu.VMEM((1, 16), jnp.int32),
            pltpu.VMEM((1, 16), jnp.int32),
            pltpu.VMEM((16, _P), jnp.float32),
            pltpu.VMEM_SHARED((_N, _P), jnp.float32),
        ],
        compiler_params=cp,
    )
    return f(edge_index)


def _pool_body(adj_ref, s_ref, ssm_ref, oadj_ref, loss_ref):
    f32 = jnp.float32
    s = s_ref[...][0]
    m = jnp.max(s, axis=1, keepdims=True)
    e = jnp.exp(s - m)
    ssm = e / jnp.sum(e, axis=1, keepdims=True)
    ssm_ref[...] = ssm

    adj = adj_ref[...][0]
    sta = jax.lax.dot_general(ssm, adj, (((0,), (0,)), ((), ())),
                              preferred_element_type=f32)
    oadj = jnp.dot(sta, ssm, preferred_element_type=f32)
    oadj_ref[...] = oadj.reshape(1, _K, _K)

    sst = jax.lax.dot_general(ssm, ssm, (((1,), (1,)), ((), ())),
                              preferred_element_type=f32)
    link = adj - sst
    ll = jnp.sqrt(jnp.sum(link * link, keepdims=True)) / (_N * _N)
    ent = -jnp.sum(ssm * jnp.log(ssm + _EPS), keepdims=True) / _N
    loss_ref[...] = jnp.concatenate([ll, ent], axis=1)


def _conv_body(a_ref, x_ref, ssm_ref,
               w1r_ref, b1_ref, w1s_ref,
               w2r_ref, b2_ref, w2s_ref,
               w3r_ref, b3_ref, w3s_ref,
               out_ref):
    f32 = jnp.float32
    ap = a_ref[...]
    a_mat = (ap[0] + ap[1])[:, 0:_N]

    def layer(h, wr, b, wroot):
        rel = jax.lax.dot_general(h, wr, (((1,), (1,)), ((), ())),
                                  preferred_element_type=f32)
        agg = jnp.dot(a_mat, rel, preferred_element_type=f32)
        root = jax.lax.dot_general(h, wroot, (((1,), (1,)), ((), ())),
                                   preferred_element_type=f32)
        return jnp.maximum(agg + root + b, 0.0)

    h = layer(x_ref[...], w1r_ref[...], b1_ref[...], w1s_ref[...])
    h = layer(h, w2r_ref[...], b2_ref[...], w2s_ref[...])
    h = layer(h, w3r_ref[...], b3_ref[...], w3s_ref[...])

    ssm = ssm_ref[...]
    out = jax.lax.dot_general(ssm, h, (((0,), (0,)), ((), ())),
                              preferred_element_type=f32)
    out_ref[...] = out.reshape(1, _K, _H)


def kernel(x, edge_index, pos, adj, s,
           W1_rel, b1, W1_root, W2_rel, b2, W2_root, W3_rel, b3, W3_root):
    a_pad = _sc_build_a(edge_index)

    ssm, out_adj, losses = pl.pallas_call(
        _pool_body,
        out_shape=[
            jax.ShapeDtypeStruct((_N, _K), jnp.float32),
            jax.ShapeDtypeStruct((1, _K, _K), jnp.float32),
            jax.ShapeDtypeStruct((1, 2), jnp.float32),
        ],
    )(adj, s)

    out, = pl.pallas_call(
        _conv_body,
        out_shape=[jax.ShapeDtypeStruct((1, _K, _H), jnp.float32)],
    )(a_pad, x, ssm,
      W1_rel, b1, W1_root, W2_rel, b2, W2_root, W3_rel, b3, W3_root)
    return (out, out_adj, losses[0, 0], losses[0, 1], pos)
